# Initial kernel scaffold; baseline (speedup 1.0000x reference)
import jax
import jax.numpy as jnp
from jax.experimental import pallas as pl


def kernel(x, edge_index, edge_weight, W1, We1, as1, ad1, ae1, b1, W2, We2, as2, ad2, ae2, b2):
    raise NotImplementedError("write your pallas kernel here")


# speedup vs baseline: 14.5975x; 14.5975x over previous
import functools

import jax
import jax.numpy as jnp
from jax import lax
from jax.experimental import pallas as pl
from jax.experimental.pallas import tpu as pltpu
from jax.experimental.pallas import tpu_sc as plsc

N = 10000
E = 320000
D = 128
NC = 2
NS = 16
NW = NC * NS
K = 128
CH = 80
EPT = K * CH
EP = EPT * NW
NP = 10240
NH = NP // 2
SH = NH + 8
TRH = NH // NS
DR = 64
DC = TRH // DR


def _prep_body(x_ref, w_ref, av_ref, h_ref, aux_ref):
    h = lax.dot_general(x_ref[...], w_ref[...], (((1,), (1,)), ((), ())),
                        preferred_element_type=jnp.float32)
    h_ref[...] = h
    aux_ref[...] = jnp.dot(h, av_ref[...], preferred_element_type=jnp.float32)


def _prep(x, W, Av):
    return pl.pallas_call(
        _prep_body,
        grid=(50,),
        in_specs=[
            pl.BlockSpec((200, D), lambda i: (i, 0)),
            pl.BlockSpec((D, D), lambda i: (0, 0)),
            pl.BlockSpec((D, 8), lambda i: (0, 0)),
        ],
        out_specs=[
            pl.BlockSpec((200, D), lambda i: (i, 0)),
            pl.BlockSpec((200, 8), lambda i: (i, 0)),
        ],
        out_shape=[
            jax.ShapeDtypeStruct((N, D), jnp.float32),
            jax.ShapeDtypeStruct((N, 8), jnp.float32),
        ],
    )(x, W, Av)


def _cew_body(ew_ref, we1_ref, ae1_ref, we2_ref, ae2_ref, o1_ref, o2_ref):
    c1 = jnp.sum(we1_ref[...] * ae1_ref[...])
    c2 = jnp.sum(we2_ref[...] * ae2_ref[...])
    ew = ew_ref[...]
    o1_ref[...] = ew * c1
    o2_ref[...] = ew * c2


def _cew(ew2d, we1, ae1, we2, ae2):
    full = pl.BlockSpec((1, D), lambda i: (0, 0))
    blk = pl.BlockSpec((320, K), lambda i: (i, 0))
    return pl.pallas_call(
        _cew_body,
        grid=(EP // K // 320,),
        in_specs=[blk, full, full, full, full],
        out_specs=[blk, blk],
        out_shape=[jax.ShapeDtypeStruct((EP // K, K), jnp.float32)] * 2,
    )(ew2d, we1, ae1, we2, ae2)


def _sca_body(asrc_hbm, adst_hbm, src_hbm, dst_hbm, ew_hbm,
              p_out, den_out, deg_out, cews_out,
              asrc_v, adst_v, src_v, dst_v, ew_v, den_v, deg_v, cews_v):
    cid = lax.axis_index("c")
    sid = lax.axis_index("s")
    wid = cid * NS + sid

    pltpu.sync_copy(asrc_hbm, asrc_v)
    pltpu.sync_copy(adst_hbm, adst_v)
    pltpu.sync_copy(src_hbm.at[wid], src_v)
    pltpu.sync_copy(dst_hbm.at[wid], dst_v)
    pltpu.sync_copy(ew_hbm.at[wid], ew_v)

    z16 = jnp.zeros((16,), jnp.float32)

    def _zero_acc(i, carry):
        sl = pl.ds(i * 16, 16)
        den_v[sl] = z16
        deg_v[sl] = z16
        cews_v[sl] = z16
        return carry

    lax.fori_loop(0, N // 16, _zero_acc, 0)

    def _phase_a(j, carry):
        for g in range(8):
            sl = pl.ds(g * 16, 16)
            sg = src_v[j, sl]
            dg = dst_v[j, sl]
            eg = ew_v[j, sl]
            a = (plsc.load_gather(asrc_v, [sg])
                 + plsc.load_gather(adst_v, [dg])
                 + eg)
            a = jnp.where(a >= 0.0, a, 0.2 * a)
            p = jnp.exp(a)
            eidx = (wid * EPT + j * K + g * 16
                    + lax.broadcasted_iota(jnp.int32, (16,), 0))
            valid = eidx < E
            p = jnp.where(valid, p, 0.0)
            ew_v[j, sl] = p
            plsc.addupdate_scatter(den_v, [dg], p)
            plsc.addupdate_scatter(deg_v, [dg], jnp.where(valid, 1.0, 0.0))
            plsc.addupdate_scatter(cews_v, [dg], jnp.where(valid, eg, 0.0))
        return carry

    lax.fori_loop(0, CH, _phase_a, 0)

    pltpu.sync_copy(ew_v, p_out.at[wid])
    pltpu.sync_copy(den_v, den_out.at[pl.ds(wid * N, N)])
    pltpu.sync_copy(deg_v, deg_out.at[pl.ds(wid * N, N)])
    pltpu.sync_copy(cews_v, cews_out.at[pl.ds(wid * N, N)])


def _sc_a(asrc, adst, srcp, dstp, ewp):
    mesh = plsc.VectorSubcoreMesh(core_axis_name="c", subcore_axis_name="s")
    return pl.kernel(
        _sca_body,
        out_type=[
            jax.ShapeDtypeStruct((NW, CH, K), jnp.float32),
            jax.ShapeDtypeStruct((NW * N,), jnp.float32),
            jax.ShapeDtypeStruct((NW * N,), jnp.float32),
            jax.ShapeDtypeStruct((NW * N,), jnp.float32),
        ],
        mesh=mesh,
        compiler_params=pltpu.CompilerParams(needs_layout_passes=False),
        scratch_types=[
            pltpu.VMEM((N,), jnp.float32),
            pltpu.VMEM((N,), jnp.float32),
            pltpu.VMEM((CH, K), jnp.int32),
            pltpu.VMEM((CH, K), jnp.int32),
            pltpu.VMEM((CH, K), jnp.float32),
            pltpu.VMEM((N,), jnp.float32),
            pltpu.VMEM((N,), jnp.float32),
            pltpu.VMEM((N,), jnp.float32),
        ],
    )(asrc, adst, srcp, dstp, ewp)


def _scb_body(h_hbm, src_hbm, dst_hbm, p_hbm, s_out,
              src_v, dst_v, p_v, rows_v, s_sh):
    cid = lax.axis_index("c")
    sid = lax.axis_index("s")
    wid = cid * NS + sid

    pltpu.sync_copy(src_hbm.at[wid], src_v)
    pltpu.sync_copy(dst_hbm.at[wid], dst_v)
    pltpu.sync_copy(p_hbm.at[wid], p_v)

    z16 = jnp.zeros((16,), jnp.float32)

    def _zero_rows(i, carry):
        for q in range(8):
            rows_v[i, pl.ds(q * 16, 16)] = z16
        return carry

    lax.fori_loop(0, K, _zero_rows, 0)
    for k in range(NP // NS // K):
        pltpu.sync_copy(rows_v, s_sh.at[pl.ds(sid * (NP // NS) + k * K, K)])
    plsc.subcore_barrier()

    def _phase_b(j, carry):
        pltpu.sync_copy(h_hbm.at[src_v.at[j]], rows_v)

        def _scale(rb, c2):
            p16 = p_v[j, pl.ds(rb * 16, 16)]
            for r in range(16):
                pr = p16[r]
                row = rb * 16 + r
                for q in range(8):
                    qsl = pl.ds(q * 16, 16)
                    rows_v[row, qsl] = rows_v[row, qsl] * pr
            return c2

        lax.fori_loop(0, K // 16, _scale, 0)
        pltpu.sync_copy(rows_v, s_sh.at[dst_v.at[j]], add=True)
        return carry

    lax.fori_loop(0, CH, _phase_b, 0)
    plsc.subcore_barrier()

    for k in range(NP // NS // K):
        sl = pl.ds(sid * (NP // NS) + k * K, K)
        pltpu.sync_copy(s_sh.at[sl], rows_v)
        pltpu.sync_copy(rows_v, s_out.at[cid, sl])


def _sc_b(h, srcp, dstp, p):
    mesh = plsc.VectorSubcoreMesh(core_axis_name="c", subcore_axis_name="s")
    return pl.kernel(
        _scb_body,
        out_type=jax.ShapeDtypeStruct((NC, NP, D), jnp.float32),
        mesh=mesh,
        compiler_params=pltpu.CompilerParams(needs_layout_passes=False),
        scratch_types=[
            pltpu.VMEM((CH, K), jnp.int32),
            pltpu.VMEM((CH, K), jnp.int32),
            pltpu.VMEM((CH, K), jnp.float32),
            pltpu.VMEM((K, D), jnp.float32),
            pltpu.VMEM_SHARED((NP, D), jnp.float32),
        ],
    )(h, srcp, dstp, p)


def _fin_body(project, sa_ref, sb_ref, den_ref, deg_ref, cews_ref, h_ref,
              aux_ref, b_ref, *rest):
    if project:
        w_ref, av_ref, h2_ref, aux2_ref = rest
    else:
        (out_ref,) = rest
    ones = jnp.ones((NW, 1), jnp.float32)
    cdims = (((1,), (0,)), ((), ()))
    den = lax.dot_general(den_ref[...], ones, cdims,
                          preferred_element_type=jnp.float32)
    deg = lax.dot_general(deg_ref[...], ones, cdims,
                          preferred_element_type=jnp.float32)
    cews = lax.dot_general(cews_ref[...], ones, cdims,
                           preferred_element_type=jnp.float32)
    la = cews / jnp.maximum(deg, 1.0)
    t = aux_ref[..., 0:1] + aux_ref[..., 1:2] + la
    t = jnp.where(t >= 0.0, t, 0.2 * t)
    ps = jnp.exp(t)
    s = sa_ref[...] + sb_ref[...]
    h = h_ref[...]
    o = (s + ps * h) / (den + ps + 1e-16) + b_ref[...]
    if project:
        o = jnp.maximum(o, 0.0)
        h2 = lax.dot_general(o, w_ref[...], (((1,), (1,)), ((), ())),
                             preferred_element_type=jnp.float32)
        h2_ref[...] = h2
        aux2_ref[...] = jnp.dot(h2, av_ref[...],
                                preferred_element_type=jnp.float32)
    else:
        out_ref[...] = o


def _finalize(sa, sb, den, deg, cews, h, aux, b, W=None, Av=None):
    project = W is not None
    full = lambda r, c: pl.BlockSpec((r, c), lambda i: (0, 0))
    blk = lambda r, c: pl.BlockSpec((r, c), lambda i: (i, 0))
    colb = pl.BlockSpec((200, NW), lambda i: (i, 0))
    in_specs = [blk(200, D), blk(200, D), colb, colb, colb, blk(200, D),
                blk(200, 8), full(1, D)]
    args = [sa, sb, den, deg, cews, h, aux, b]
    if project:
        in_specs += [full(D, D), full(D, 8)]
        args += [W, Av]
        out_specs = [blk(200, D), blk(200, 8)]
        out_shape = [jax.ShapeDtypeStruct((N, D), jnp.float32),
                     jax.ShapeDtypeStruct((N, 8), jnp.float32)]
    else:
        out_specs = [blk(200, D)]
        out_shape = [jax.ShapeDtypeStruct((N, D), jnp.float32)]
    res = pl.pallas_call(
        functools.partial(_fin_body, project),
        grid=(50,),
        in_specs=in_specs,
        out_specs=out_specs,
        out_shape=out_shape,
    )(*args)
    return res


def kernel(x, edge_index, edge_weight, W1, We1, as1, ad1, ae1, b1,
           W2, We2, as2, ad2, ae2, b2):
    src = edge_index[0]
    dst = edge_index[1]
    ew = edge_weight[:, 0]
    pad = EP - E
    srcp = jnp.pad(src, (0, pad)).reshape(NW, CH, K)
    dstp = jnp.pad(dst, (0, pad)).reshape(NW, CH, K)
    ew2d = jnp.pad(ew, (0, pad)).reshape(EP // K, K)
    cew1, cew2 = _cew(ew2d, We1.reshape(1, D), ae1.reshape(1, D),
                      We2.reshape(1, D), ae2.reshape(1, D))
    cewp1 = cew1.reshape(NW, CH, K)
    cewp2 = cew2.reshape(NW, CH, K)
    zc = jnp.zeros((D, 6), jnp.float32)
    Av1 = jnp.concatenate([as1.reshape(D, 1), ad1.reshape(D, 1), zc], axis=1)
    Av2 = jnp.concatenate([as2.reshape(D, 1), ad2.reshape(D, 1), zc], axis=1)

    h1, aux1 = _prep(x, W1, Av1)
    p1, den1, deg1, cews1 = _sc_a(aux1[:, 0], aux1[:, 1], srcp, dstp, cewp1)
    s1 = _sc_b(h1, srcp, dstp, p1)
    h2, aux2 = _finalize(s1[0, :N], s1[1, :N],
                         den1.reshape(NW, N).T, deg1.reshape(NW, N).T,
                         cews1.reshape(NW, N).T, h1, aux1,
                         b1.reshape(1, D), W2, Av2)
    p2, den2, deg2, cews2 = _sc_a(aux2[:, 0], aux2[:, 1], srcp, dstp, cewp2)
    s2 = _sc_b(h2, srcp, dstp, p2)
    out = _finalize(s2[0, :N], s2[1, :N],
                    den2.reshape(NW, N).T, deg2.reshape(NW, N).T,
                    cews2.reshape(NW, N).T, h2, aux2,
                    b2.reshape(1, D))
    return out[0]


# speedup vs baseline: 15.1002x; 1.0344x over previous
import functools

import jax
import jax.numpy as jnp
from jax import lax
from jax.experimental import pallas as pl
from jax.experimental.pallas import tpu as pltpu
from jax.experimental.pallas import tpu_sc as plsc

N = 10000
E = 320000
D = 128
NC = 2
NS = 16
NW = NC * NS
K = 128
CH = 80
EPT = K * CH
EP = EPT * NW
NP = 10240
RPT = NP // NS


def _prep_body(x_ref, w_ref, av_ref, h_ref, aux_ref):
    h = lax.dot_general(x_ref[...], w_ref[...], (((1,), (1,)), ((), ())),
                        preferred_element_type=jnp.float32)
    h_ref[...] = h
    aux_ref[...] = jnp.dot(h, av_ref[...], preferred_element_type=jnp.float32)


def _prep(x, W, Av):
    return pl.pallas_call(
        _prep_body,
        grid=(50,),
        in_specs=[
            pl.BlockSpec((200, D), lambda i: (i, 0)),
            pl.BlockSpec((D, D), lambda i: (0, 0)),
            pl.BlockSpec((D, 8), lambda i: (0, 0)),
        ],
        out_specs=[
            pl.BlockSpec((200, D), lambda i: (i, 0)),
            pl.BlockSpec((200, 8), lambda i: (i, 0)),
        ],
        out_shape=[
            jax.ShapeDtypeStruct((N, D), jnp.float32),
            jax.ShapeDtypeStruct((N, 8), jnp.float32),
        ],
    )(x, W, Av)


def _cew_body(ew_ref, we1_ref, ae1_ref, we2_ref, ae2_ref, o1_ref, o2_ref):
    c1 = jnp.sum(we1_ref[...] * ae1_ref[...])
    c2 = jnp.sum(we2_ref[...] * ae2_ref[...])
    ew = ew_ref[...]
    o1_ref[...] = ew * c1
    o2_ref[...] = ew * c2


def _cew(ew2d, we1, ae1, we2, ae2):
    full = pl.BlockSpec((1, D), lambda i: (0, 0))
    blk = pl.BlockSpec((320, K), lambda i: (i, 0))
    return pl.pallas_call(
        _cew_body,
        grid=(EP // K // 320,),
        in_specs=[blk, full, full, full, full],
        out_specs=[blk, blk],
        out_shape=[jax.ShapeDtypeStruct((EP // K, K), jnp.float32)] * 2,
    )(ew2d, we1, ae1, we2, ae2)


def _sca_body(asrc_hbm, adst_hbm, src_hbm, dst_hbm, ew_hbm,
              p_out, den_out, deg_out, cews_out,
              asrc_v, adst_v, src_v, dst_v, ew_v, den_v, deg_v, cews_v):
    cid = lax.axis_index("c")
    sid = lax.axis_index("s")
    wid = cid * NS + sid

    pltpu.sync_copy(asrc_hbm, asrc_v)
    pltpu.sync_copy(adst_hbm, adst_v)
    pltpu.sync_copy(src_hbm.at[wid], src_v)
    pltpu.sync_copy(dst_hbm.at[wid], dst_v)
    pltpu.sync_copy(ew_hbm.at[wid], ew_v)

    z16 = jnp.zeros((16,), jnp.float32)

    def _zero_acc(i, carry):
        sl = pl.ds(i * 16, 16)
        den_v[sl] = z16
        deg_v[sl] = z16
        cews_v[sl] = z16
        return carry

    lax.fori_loop(0, N // 16, _zero_acc, 0)

    def _phase_a(j, carry):
        for g in range(8):
            sl = pl.ds(g * 16, 16)
            sg = src_v[j, sl]
            dg = dst_v[j, sl]
            eg = ew_v[j, sl]
            a = (plsc.load_gather(asrc_v, [sg])
                 + plsc.load_gather(adst_v, [dg])
                 + eg)
            a = jnp.where(a >= 0.0, a, 0.2 * a)
            p = jnp.exp(a)
            eidx = (wid * EPT + j * K + g * 16
                    + lax.broadcasted_iota(jnp.int32, (16,), 0))
            valid = eidx < E
            p = jnp.where(valid, p, 0.0)
            ew_v[j, sl] = p
            plsc.addupdate_scatter(den_v, [dg], p)
            plsc.addupdate_scatter(deg_v, [dg], jnp.where(valid, 1.0, 0.0))
            plsc.addupdate_scatter(cews_v, [dg], jnp.where(valid, eg, 0.0))
        return carry

    lax.fori_loop(0, CH, _phase_a, 0)

    pltpu.sync_copy(ew_v, p_out.at[wid])
    pltpu.sync_copy(den_v, den_out.at[pl.ds(wid * N, N)])
    pltpu.sync_copy(deg_v, deg_out.at[pl.ds(wid * N, N)])
    pltpu.sync_copy(cews_v, cews_out.at[pl.ds(wid * N, N)])


def _sc_a(asrc, adst, srcp, dstp, ewp):
    mesh = plsc.VectorSubcoreMesh(core_axis_name="c", subcore_axis_name="s")
    return pl.kernel(
        _sca_body,
        out_type=[
            jax.ShapeDtypeStruct((NW, CH, K), jnp.float32),
            jax.ShapeDtypeStruct((NW * N,), jnp.float32),
            jax.ShapeDtypeStruct((NW * N,), jnp.float32),
            jax.ShapeDtypeStruct((NW * N,), jnp.float32),
        ],
        mesh=mesh,
        compiler_params=pltpu.CompilerParams(needs_layout_passes=False),
        scratch_types=[
            pltpu.VMEM((N,), jnp.float32),
            pltpu.VMEM((N,), jnp.float32),
            pltpu.VMEM((CH, K), jnp.int32),
            pltpu.VMEM((CH, K), jnp.int32),
            pltpu.VMEM((CH, K), jnp.float32),
            pltpu.VMEM((N,), jnp.float32),
            pltpu.VMEM((N,), jnp.float32),
            pltpu.VMEM((N,), jnp.float32),
        ],
    )(asrc, adst, srcp, dstp, ewp)


def _scb_body(h_hbm, src_hbm, dst_hbm, p_hbm, s_out,
              src_v, dst_v, p_v, rows_v, s_sh):
    cid = lax.axis_index("c")
    sid = lax.axis_index("s")
    wid = cid * NS + sid

    pltpu.sync_copy(src_hbm.at[wid], src_v)
    pltpu.sync_copy(dst_hbm.at[wid], dst_v)
    pltpu.sync_copy(p_hbm.at[wid], p_v)

    z16 = jnp.zeros((16,), jnp.float32)

    def _zero_rows(i, carry):
        for q in range(8):
            rows_v[i, pl.ds(q * 16, 16)] = z16
        return carry

    lax.fori_loop(0, K, _zero_rows, 0)
    for k in range(RPT // K):
        pltpu.sync_copy(rows_v, s_sh.at[pl.ds(sid * RPT + k * K, K)])
    plsc.subcore_barrier()

    def _phase_b(j, carry):
        pltpu.sync_copy(h_hbm.at[src_v.at[j]], rows_v)

        @plsc.parallel_loop(0, K // 16, unroll=2)
        def _scale(rb):
            p16 = p_v[j, pl.ds(rb * 16, 16)]
            for r in range(16):
                pr = p16[r]
                row = rb * 16 + r
                for q in range(8):
                    qsl = pl.ds(q * 16, 16)
                    rows_v[row, qsl] = rows_v[row, qsl] * pr
        pltpu.sync_copy(rows_v, s_sh.at[dst_v.at[j]], add=True)
        return carry

    lax.fori_loop(0, CH, _phase_b, 0)
    plsc.subcore_barrier()

    for k in range(RPT // K):
        sl = pl.ds(sid * RPT + k * K, K)
        pltpu.sync_copy(s_sh.at[sl], rows_v)
        pltpu.sync_copy(rows_v, s_out.at[cid, sl])


def _sc_b(h, srcp, dstp, p):
    mesh = plsc.VectorSubcoreMesh(core_axis_name="c", subcore_axis_name="s")
    return pl.kernel(
        _scb_body,
        out_type=jax.ShapeDtypeStruct((NC, NP, D), jnp.float32),
        mesh=mesh,
        compiler_params=pltpu.CompilerParams(needs_layout_passes=False),
        scratch_types=[
            pltpu.VMEM((CH, K), jnp.int32),
            pltpu.VMEM((CH, K), jnp.int32),
            pltpu.VMEM((CH, K), jnp.float32),
            pltpu.VMEM((K, D), jnp.float32),
            pltpu.VMEM_SHARED((NP, D), jnp.float32),
        ],
    )(h, srcp, dstp, p)


def _fin_body(project, sa_ref, sb_ref, den_ref, deg_ref, cews_ref, h_ref,
              aux_ref, b_ref, *rest):
    if project:
        w_ref, av_ref, h2_ref, aux2_ref = rest
    else:
        (out_ref,) = rest
    ones = jnp.ones((NW, 1), jnp.float32)
    cdims = (((1,), (0,)), ((), ()))
    den = lax.dot_general(den_ref[...], ones, cdims,
                          preferred_element_type=jnp.float32)
    deg = lax.dot_general(deg_ref[...], ones, cdims,
                          preferred_element_type=jnp.float32)
    cews = lax.dot_general(cews_ref[...], ones, cdims,
                           preferred_element_type=jnp.float32)
    la = cews / jnp.maximum(deg, 1.0)
    t = aux_ref[..., 0:1] + aux_ref[..., 1:2] + la
    t = jnp.where(t >= 0.0, t, 0.2 * t)
    ps = jnp.exp(t)
    s = sa_ref[...] + sb_ref[...]
    h = h_ref[...]
    o = (s + ps * h) / (den + ps + 1e-16) + b_ref[...]
    if project:
        o = jnp.maximum(o, 0.0)
        h2 = lax.dot_general(o, w_ref[...], (((1,), (1,)), ((), ())),
                             preferred_element_type=jnp.float32)
        h2_ref[...] = h2
        aux2_ref[...] = jnp.dot(h2, av_ref[...],
                                preferred_element_type=jnp.float32)
    else:
        out_ref[...] = o


def _finalize(sa, sb, den, deg, cews, h, aux, b, W=None, Av=None):
    project = W is not None
    full = lambda r, c: pl.BlockSpec((r, c), lambda i: (0, 0))
    blk = lambda r, c: pl.BlockSpec((r, c), lambda i: (i, 0))
    colb = pl.BlockSpec((200, NW), lambda i: (i, 0))
    in_specs = [blk(200, D), blk(200, D), colb, colb, colb, blk(200, D),
                blk(200, 8), full(1, D)]
    args = [sa, sb, den, deg, cews, h, aux, b]
    if project:
        in_specs += [full(D, D), full(D, 8)]
        args += [W, Av]
        out_specs = [blk(200, D), blk(200, 8)]
        out_shape = [jax.ShapeDtypeStruct((N, D), jnp.float32),
                     jax.ShapeDtypeStruct((N, 8), jnp.float32)]
    else:
        out_specs = [blk(200, D)]
        out_shape = [jax.ShapeDtypeStruct((N, D), jnp.float32)]
    res = pl.pallas_call(
        functools.partial(_fin_body, project),
        grid=(50,),
        in_specs=in_specs,
        out_specs=out_specs,
        out_shape=out_shape,
    )(*args)
    return res


def kernel(x, edge_index, edge_weight, W1, We1, as1, ad1, ae1, b1,
           W2, We2, as2, ad2, ae2, b2):
    src = edge_index[0]
    dst = edge_index[1]
    ew = edge_weight[:, 0]
    pad = EP - E
    srcp = jnp.pad(src, (0, pad)).reshape(NW, CH, K)
    dstp = jnp.pad(dst, (0, pad)).reshape(NW, CH, K)
    ew2d = jnp.pad(ew, (0, pad)).reshape(EP // K, K)
    cew1, cew2 = _cew(ew2d, We1.reshape(1, D), ae1.reshape(1, D),
                      We2.reshape(1, D), ae2.reshape(1, D))
    cewp1 = cew1.reshape(NW, CH, K)
    cewp2 = cew2.reshape(NW, CH, K)
    zc = jnp.zeros((D, 6), jnp.float32)
    Av1 = jnp.concatenate([as1.reshape(D, 1), ad1.reshape(D, 1), zc], axis=1)
    Av2 = jnp.concatenate([as2.reshape(D, 1), ad2.reshape(D, 1), zc], axis=1)

    h1, aux1 = _prep(x, W1, Av1)
    p1, den1, deg1, cews1 = _sc_a(aux1[:, 0], aux1[:, 1], srcp, dstp, cewp1)
    s1 = _sc_b(h1, srcp, dstp, p1)
    h2, aux2 = _finalize(s1[0, :N], s1[1, :N],
                         den1.reshape(NW, N).T, deg1.reshape(NW, N).T,
                         cews1.reshape(NW, N).T, h1, aux1,
                         b1.reshape(1, D), W2, Av2)
    p2, den2, deg2, cews2 = _sc_a(aux2[:, 0], aux2[:, 1], srcp, dstp, cewp2)
    s2 = _sc_b(h2, srcp, dstp, p2)
    out = _finalize(s2[0, :N], s2[1, :N],
                    den2.reshape(NW, N).T, deg2.reshape(NW, N).T,
                    cews2.reshape(NW, N).T, h2, aux2,
                    b2.reshape(1, D))
    return out[0]
